# Initial kernel scaffold; baseline (speedup 1.0000x reference)
import jax
import jax.numpy as jnp
from jax.experimental import pallas as pl


def kernel(last_embs, edge_index, edge_values):
    raise NotImplementedError("write your pallas kernel here")


# speedup vs baseline: 17.5093x; 17.5093x over previous
import functools

import jax
import jax.numpy as jnp
from jax import lax
from jax.experimental import pallas as pl
from jax.experimental.pallas import tpu as pltpu
from jax.experimental.pallas import tpu_sc as plsc

B = 4
N = 10000
D = 128
E = 320000

NC = 2
NT = 16
EPT = E // NT
CH = 80
NCHUNK = EPT // CH
RPT = 640
N_PAD = NT * RPT
NV = D // 16

_mesh = plsc.VectorSubcoreMesh(
    core_axis_name="c", subcore_axis_name="s", num_cores=NC, num_subcores=NT
)


@functools.partial(
    pl.kernel,
    out_type=jax.ShapeDtypeStruct((B, N, D), jnp.float32),
    mesh=_mesh,
    scratch_types=[
        pltpu.VMEM((NCHUNK, CH), jnp.float32),
        pltpu.VMEM((CH,), jnp.int32),
        pltpu.VMEM((CH,), jnp.int32),
        pltpu.VMEM((CH, D), jnp.float32),
        pltpu.VMEM_SHARED((N_PAD, D), jnp.float32),
        pltpu.SemaphoreType.DMA,
    ],
)
def _aggregate(emb_hbm, col_hbm, row_hbm, val_hbm, out_hbm,
               valv, colv, rowv, rows, acc, sem):
    c = lax.axis_index("c")
    s = lax.axis_index("s")

    zvec = jnp.zeros((16,), jnp.float32)

    def zero_row(e, carry):
        for q in range(NV):
            rows[e, pl.ds(q * 16, 16)] = zvec
        return carry

    lax.fori_loop(0, CH, zero_row, 0)

    for i in range(B // NC):
        b = c * (B // NC) + i

        for k in range(RPT // CH):
            pltpu.sync_copy(rows, acc.at[pl.ds(s * RPT + k * CH, CH)])

        pltpu.sync_copy(val_hbm.at[b, s], valv)

        plsc.subcore_barrier()

        def chunk(j, carry):
            pltpu.sync_copy(col_hbm.at[b, s, j], colv)
            pltpu.sync_copy(row_hbm.at[b, s, j], rowv)
            pltpu.async_copy(emb_hbm.at[colv], rows, sem).wait()

            def grp(g, gcarry):
                v16 = valv[j, pl.ds(g * 16, 16)]
                for k in range(16):
                    e = g * 16 + k
                    v = v16[k]
                    for q in range(NV):
                        sl = pl.ds(q * 16, 16)
                        rows[e, sl] = rows[e, sl] * v
                return gcarry

            lax.fori_loop(0, CH // 16, grp, 0)

            pltpu.sync_copy(rows, acc.at[rowv], add=True)
            return carry

        lax.fori_loop(0, NCHUNK, chunk, 0)

        plsc.subcore_barrier()

        @pl.when(s < NT - 1)
        def _write_full():
            sl = pl.ds(s * RPT, RPT)
            pltpu.sync_copy(acc.at[sl], out_hbm.at[b, sl])

        @pl.when(s == NT - 1)
        def _write_tail():
            sl = pl.ds((NT - 1) * RPT, N - (NT - 1) * RPT)
            pltpu.sync_copy(acc.at[sl], out_hbm.at[b, sl])

        plsc.subcore_barrier()

        if i + 1 < B // NC:
            lax.fori_loop(0, CH, zero_row, 0)


def kernel(last_embs, edge_index, edge_values):
    ei = edge_index.astype(jnp.int32)
    col = ei[:, 1, :] + (jnp.arange(B, dtype=jnp.int32) * N)[:, None]
    row = ei[:, 0, :]
    emb2 = last_embs.reshape(B * N, D)
    col4 = col.reshape(B, NT, NCHUNK, CH)
    row4 = row.reshape(B, NT, NCHUNK, CH)
    val4 = edge_values.reshape(B, NT, NCHUNK, CH)
    return _aggregate(emb2, col4, row4, val4)
